# baseline (speedup 1.0000x reference)
import jax
import jax.numpy as jnp
from jax.experimental import pallas as pl
from jax.experimental.pallas import tpu as pltpu

_NT = 16384
_D = 2048
_E = 16
_BT = 512
_STEPS = _NT // _BT


def _gating_kernel(x_ref, w_ref, gate_ref, idx_ref, loss_ref, acc_ref):
    step = pl.program_id(0)
    x = x_ref[...]
    w = w_ref[...]
    logits = jax.lax.dot_general(
        x, w, (((1,), (1,)), ((), ())), preferred_element_type=jnp.float32
    )
    iota = jax.lax.broadcasted_iota(jnp.int32, logits.shape, 1)
    m1 = jnp.max(logits, axis=1, keepdims=True)
    is1 = logits == m1
    idx1 = jnp.min(jnp.where(is1, iota, _E), axis=1, keepdims=True)
    masked = jnp.where(iota == idx1, -jnp.inf, logits)
    m2 = jnp.max(masked, axis=1, keepdims=True)
    idx2 = jnp.min(jnp.where(masked == m2, iota, _E), axis=1, keepdims=True)
    t = jnp.exp(m2 - m1)
    denom = 1.0 + t
    gate_ref[...] = jnp.concatenate([1.0 / denom, t / denom], axis=1)
    idx_ref[...] = jnp.concatenate([idx1, idx2], axis=1)
    e = jnp.exp(logits - m1)
    p = e / jnp.sum(e, axis=1, keepdims=True)
    psum = jnp.sum(p, axis=0)
    csum = jnp.sum((iota == idx1).astype(jnp.float32), axis=0)
    part = jnp.stack([psum, csum])

    @pl.when(step == 0)
    def _init():
        acc_ref[...] = part

    @pl.when(step != 0)
    def _accum():
        acc_ref[...] += part

    @pl.when(step == _STEPS - 1)
    def _finalize():
        acc = acc_ref[...]
        loss = _E * jnp.sum(acc[0] * acc[1], keepdims=True) / (_NT * _NT)
        loss_ref[...] = loss.reshape(1, 1)


def kernel(x, W):
    gate, idx, loss = pl.pallas_call(
        _gating_kernel,
        grid=(_STEPS,),
        in_specs=[
            pl.BlockSpec((_BT, _D), lambda i: (i, 0)),
            pl.BlockSpec((_E, _D), lambda i: (0, 0)),
        ],
        out_specs=[
            pl.BlockSpec((_BT, 2), lambda i: (i, 0)),
            pl.BlockSpec((_BT, 2), lambda i: (i, 0)),
            pl.BlockSpec((1, 1), lambda i: (0, 0)),
        ],
        out_shape=[
            jax.ShapeDtypeStruct((_NT, 2), jnp.float32),
            jax.ShapeDtypeStruct((_NT, 2), jnp.int32),
            jax.ShapeDtypeStruct((1, 1), jnp.float32),
        ],
        scratch_shapes=[pltpu.VMEM((2, _E), jnp.float32)],
    )(x, W)
    return gate, idx, loss[0, 0]


# speedup vs baseline: 1.2162x; 1.2162x over previous
import jax
import jax.numpy as jnp
from jax.experimental import pallas as pl
from jax.experimental.pallas import tpu as pltpu

_NT = 16384
_D = 2048
_E = 16
_BT = 2048
_STEPS = _NT // _BT


def _gating_kernel(x_ref, w_ref, gate_ref, idx_ref, loss_ref, acc_ref):
    step = pl.program_id(0)
    x = x_ref[...]
    w = w_ref[...]
    logits = jax.lax.dot_general(
        x, w, (((1,), (1,)), ((), ())), preferred_element_type=jnp.float32
    )
    iota = jax.lax.broadcasted_iota(jnp.int32, logits.shape, 1)
    m1 = jnp.max(logits, axis=1, keepdims=True)
    is1 = logits == m1
    idx1 = jnp.min(jnp.where(is1, iota, _E), axis=1, keepdims=True)
    masked = jnp.where(iota == idx1, -jnp.inf, logits)
    m2 = jnp.max(masked, axis=1, keepdims=True)
    idx2 = jnp.min(jnp.where(masked == m2, iota, _E), axis=1, keepdims=True)
    t = jnp.exp(m2 - m1)
    denom = 1.0 + t
    gate_ref[...] = jnp.concatenate([1.0 / denom, t / denom], axis=1)
    idx_ref[...] = jnp.concatenate([idx1, idx2], axis=1)
    e = jnp.exp(logits - m1)
    p = e / jnp.sum(e, axis=1, keepdims=True)
    psum = jnp.sum(p, axis=0)
    csum = jnp.sum((iota == idx1).astype(jnp.float32), axis=0)
    part = jnp.stack([psum, csum])

    @pl.when(step == 0)
    def _init():
        acc_ref[...] = part

    @pl.when(step != 0)
    def _accum():
        acc_ref[...] += part

    @pl.when(step == _STEPS - 1)
    def _finalize():
        acc = acc_ref[...]
        loss = _E * jnp.sum(acc[0] * acc[1], keepdims=True) / (_NT * _NT)
        loss_ref[...] = loss.reshape(1, 1)


def kernel(x, W):
    gate, idx, loss = pl.pallas_call(
        _gating_kernel,
        grid=(_STEPS,),
        in_specs=[
            pl.BlockSpec((_BT, _D), lambda i: (i, 0)),
            pl.BlockSpec((_E, _D), lambda i: (0, 0)),
        ],
        out_specs=[
            pl.BlockSpec((_BT, 2), lambda i: (i, 0)),
            pl.BlockSpec((_BT, 2), lambda i: (i, 0)),
            pl.BlockSpec((1, 1), lambda i: (0, 0)),
        ],
        out_shape=[
            jax.ShapeDtypeStruct((_NT, 2), jnp.float32),
            jax.ShapeDtypeStruct((_NT, 2), jnp.int32),
            jax.ShapeDtypeStruct((1, 1), jnp.float32),
        ],
        scratch_shapes=[pltpu.VMEM((2, _E), jnp.float32)],
    )(x, W)
    return gate, idx, loss[0, 0]
